# Initial kernel scaffold; baseline (speedup 1.0000x reference)
import jax
import jax.numpy as jnp
from jax.experimental import pallas as pl


def kernel(x, Wg, bg, W1, b1, W2, b2):
    raise NotImplementedError("write your pallas kernel here")


# speedup vs baseline: 4.0366x; 4.0366x over previous
import functools
import math

import jax
import jax.numpy as jnp
from jax.experimental import pallas as pl
from jax.experimental.pallas import tpu as pltpu

_B, _S, _DIM, _HID, _E = 2, 2048, 1024, 4096, 8
_T = _B * _S
_EPS = 1e-6
_SQRT2 = math.sqrt(2.0)


def _gate_body(x_ref, wg_ref, bg_ref, out_ref):
    logits = jnp.dot(x_ref[...], wg_ref[...], preferred_element_type=jnp.float32)
    logits = logits + bg_ref[...]
    m = jnp.max(logits, axis=-1, keepdims=True)
    ex = jnp.exp(logits - m)
    p = ex / jnp.sum(ex, axis=-1, keepdims=True)
    ii = jax.lax.broadcasted_iota(jnp.int32, p.shape, 1)
    m0 = jnp.max(p, axis=-1, keepdims=True)
    i0 = jnp.min(jnp.where(p == m0, ii, _E), axis=-1, keepdims=True)
    oh0 = ii == i0
    p2 = jnp.where(oh0, -1.0, p)
    m1 = jnp.max(p2, axis=-1, keepdims=True)
    i1 = jnp.min(jnp.where(p2 == m1, ii, _E), axis=-1, keepdims=True)
    oh1 = ii == i1
    masked = jnp.where(oh0 | oh1, p, 0.0)
    s = jnp.sum(masked, axis=-1, keepdims=True) + _EPS
    out_ref[...] = masked / s


def _gates(x2d, Wg, bg):
    tt = 1024
    return pl.pallas_call(
        _gate_body,
        grid=(_T // tt,),
        in_specs=[
            pl.BlockSpec((tt, _DIM), lambda t: (t, 0)),
            pl.BlockSpec((_DIM, _E), lambda t: (0, 0)),
            pl.BlockSpec((1, _E), lambda t: (0, 0)),
        ],
        out_specs=pl.BlockSpec((tt, _E), lambda t: (t, 0)),
        out_shape=jax.ShapeDtypeStruct((_T, _E), jnp.float32),
    )(x2d, Wg, bg.reshape(1, _E))


def _expert_body(x_ref, g_ref, w1_ref, b1_ref, w2_ref, b2_ref, out_ref):
    e = pl.program_id(1)
    h = pl.program_id(2)

    @pl.when((e == 0) & (h == 0))
    def _init():
        out_ref[...] = jnp.dot(
            g_ref[...], b2_ref[...], preferred_element_type=jnp.float32
        )

    hmat = jnp.dot(x_ref[...], w1_ref[0], preferred_element_type=jnp.float32)
    hmat = hmat + b1_ref[0]
    hact = 0.5 * hmat * (1.0 + jax.lax.erf(hmat / _SQRT2))
    y = jnp.dot(hact, w2_ref[0], preferred_element_type=jnp.float32)
    ii = jax.lax.broadcasted_iota(jnp.int32, (g_ref.shape[0], _E), 1)
    w_col = jnp.sum(jnp.where(ii == e, g_ref[...], 0.0), axis=-1, keepdims=True)
    out_ref[...] += w_col * y


def _experts(x2d, gates, W1, b1, W2, b2):
    tt, ht = 2048, 512
    return pl.pallas_call(
        _expert_body,
        grid=(_T // tt, _E, _HID // ht),
        in_specs=[
            pl.BlockSpec((tt, _DIM), lambda t, e, h: (t, 0)),
            pl.BlockSpec((tt, _E), lambda t, e, h: (t, 0)),
            pl.BlockSpec((1, _DIM, ht), lambda t, e, h: (e, 0, h)),
            pl.BlockSpec((1, 1, ht), lambda t, e, h: (e, 0, h)),
            pl.BlockSpec((1, ht, _DIM), lambda t, e, h: (e, h, 0)),
            pl.BlockSpec((_E, _DIM), lambda t, e, h: (0, 0)),
        ],
        out_specs=pl.BlockSpec((tt, _DIM), lambda t, e, h: (t, 0)),
        out_shape=jax.ShapeDtypeStruct((_T, _DIM), jnp.float32),
    )(x2d, gates, W1, b1.reshape(_E, 1, _HID), W2, b2)


def kernel(x, Wg, bg, W1, b1, W2, b2):
    x2d = x.reshape(_T, _DIM)
    gates = _gates(x2d, Wg, bg)
    out = _experts(x2d, gates, W1, b1, W2, b2)
    return out.reshape(_B, _S, _DIM)


# speedup vs baseline: 5.6771x; 1.4064x over previous
import functools
import math

import jax
import jax.numpy as jnp
from jax import lax
from jax.experimental import pallas as pl
from jax.experimental.pallas import tpu as pltpu
from jax.experimental.pallas import tpu_sc as plsc

_B, _S, _DIM, _HID, _E = 2, 2048, 1024, 4096, 8
_T = _B * _S
_NA = 2 * _T
_EPS = 1e-6
_SQRT2 = math.sqrt(2.0)

_NW = 32
_APT = _NA // _NW
_TPT = _T // _NW
_TM = 256
_NPAD = _NA + _E * _TM
_NT = _NPAD // _TM
_NTP = 48

@functools.lru_cache(maxsize=None)
def _mesh():
    return plsc.VectorSubcoreMesh(
        core_axis_name="c", subcore_axis_name="s", num_cores=2, num_subcores=16
    )


def _gate_body(x_ref, wg_ref, bg_ref, e0_ref, e1_ref, w0_ref, w1_ref):
    logits = jnp.dot(x_ref[...], wg_ref[...], preferred_element_type=jnp.float32)
    logits = logits + bg_ref[...]
    m = jnp.max(logits, axis=-1, keepdims=True)
    ex = jnp.exp(logits - m)
    p = ex / jnp.sum(ex, axis=-1, keepdims=True)
    ii = jax.lax.broadcasted_iota(jnp.int32, p.shape, 1)
    m0 = jnp.max(p, axis=-1, keepdims=True)
    i0 = jnp.min(jnp.where(p == m0, ii, _E), axis=-1, keepdims=True)
    oh0 = ii == i0
    p2 = jnp.where(oh0, -1.0, p)
    m1 = jnp.max(p2, axis=-1, keepdims=True)
    i1 = jnp.min(jnp.where(p2 == m1, ii, _E), axis=-1, keepdims=True)
    oh1 = ii == i1
    s = m0 + m1 + _EPS
    e0_ref[...] = i0[:, 0]
    e1_ref[...] = i1[:, 0]
    w0_ref[...] = (m0 / s)[:, 0]
    w1_ref[...] = (m1 / s)[:, 0]


def _gates(x2d, Wg, bg):
    tt = 1024
    return pl.pallas_call(
        _gate_body,
        grid=(_T // tt,),
        in_specs=[
            pl.BlockSpec((tt, _DIM), lambda t: (t, 0)),
            pl.BlockSpec((_DIM, _E), lambda t: (0, 0)),
            pl.BlockSpec((1, _E), lambda t: (0, 0)),
        ],
        out_specs=[
            pl.BlockSpec((tt,), lambda t: (t,)),
            pl.BlockSpec((tt,), lambda t: (t,)),
            pl.BlockSpec((tt,), lambda t: (t,)),
            pl.BlockSpec((tt,), lambda t: (t,)),
        ],
        out_shape=[
            jax.ShapeDtypeStruct((_T,), jnp.int32),
            jax.ShapeDtypeStruct((_T,), jnp.int32),
            jax.ShapeDtypeStruct((_T,), jnp.float32),
            jax.ShapeDtypeStruct((_T,), jnp.float32),
        ],
    )(x2d, Wg, bg.reshape(1, _E))


@functools.lru_cache(maxsize=None)
def _route_dispatch_kernel():
  return functools.partial(
    pl.kernel,
    out_type=(
        jax.ShapeDtypeStruct((_NPAD, _DIM), jnp.float32),
        jax.ShapeDtypeStruct((_NPAD, 128), jnp.float32),
        jax.ShapeDtypeStruct((_NA,), jnp.int32),
        jax.ShapeDtypeStruct((_NTP,), jnp.int32),
    ),
    mesh=_mesh(),
    scratch_types=[
        pltpu.VMEM((_NA,), jnp.int32),
        pltpu.VMEM((_APT,), jnp.float32),
        pltpu.VMEM((4, 64), jnp.int32),
        pltpu.VMEM((4, 64), jnp.int32),
        pltpu.VMEM((64, _DIM), jnp.float32),
        pltpu.VMEM((64, 128), jnp.float32),
        pltpu.VMEM((_NTP,), jnp.int32),
        pltpu.SemaphoreType.DMA,
        pltpu.SemaphoreType.DMA,
    ],
    compiler_params=pltpu.CompilerParams(needs_layout_passes=False),
  )(_route_dispatch_body)


def _route_dispatch_body(e0_hbm, e1_hbm, w0_hbm, w1_hbm, x_hbm,
                    xs_hbm, ws_hbm, dst_hbm, teid_hbm,
                    evall, wv, dstv, tokv, xbuf, wbuf,
                    teidv, sem1, sem2):
    wid = lax.axis_index("s") * 2 + lax.axis_index("c")
    ab = wid * _APT
    lane = lax.iota(jnp.int32, 16)

    pltpu.sync_copy(e0_hbm, evall.at[pl.ds(0, _T)])
    pltpu.sync_copy(e1_hbm, evall.at[pl.ds(_T, _T)])

    @pl.when(wid < 16)
    def _():
        pltpu.sync_copy(w0_hbm.at[pl.ds(ab, _APT)], wv)

    @pl.when(wid >= 16)
    def _():
        pltpu.sync_copy(w1_hbm.at[pl.ds(ab - _T, _APT)], wv)

    lim = ab // 16
    zz = jnp.zeros((16,), jnp.int32)

    def _hist(j, carry):
        accs = list(carry)
        v = evall[pl.ds(j * 16, 16)]
        sel = jnp.where(j < lim, 1, 0)
        for e in range(_E):
            mme = jnp.where(v == e, 1, 0)
            accs[e] = accs[e] + mme
            accs[_E + e] = accs[_E + e] + mme * sel
        return tuple(accs)

    accs = lax.fori_loop(0, _NA // 16, _hist, (zz,) * (2 * _E))
    total = zz
    basem = zz
    for e in range(_E):
        total = total + jnp.where(lane == e, jnp.sum(accs[e], axis=0), 0)
        basem = basem + jnp.where(lane == e, jnp.sum(accs[_E + e], axis=0), 0)
    cap = ((total + (_TM - 1)) >> 8) << 8
    cum = plsc.cumsum(cap)
    start = cum - cap
    basem = basem + start
    base_s = [jnp.max(jnp.where(lane == e, basem, 0), axis=0) for e in range(_E)]
    start_s = [jnp.max(jnp.where(lane == e, start, 0), axis=0) for e in range(_E)]

    for i in range(_APT // 16):
        v = evall[pl.ds(ab + i * 16, 16)]
        acc = jnp.zeros((16,), jnp.int32)
        for e in range(_E):
            mm = v == e
            ones = jnp.where(mm, 1, 0)
            r = plsc.cumsum(ones)
            acc = jnp.where(mm, base_s[e] + r - 1, acc)
            base_s[e] = base_s[e] + jnp.max(r, axis=0)
        dstv[i // 4, pl.ds((i % 4) * 16, 16)] = jnp.clip(acc, 0, _NPAD - 1)
        tok = lane + (ab + i * 16) - jnp.where(wid >= 16, _T, 0)
        tokv[i // 4, pl.ds((i % 4) * 16, 16)] = jnp.clip(tok, 0, _T - 1)

    for c in range(4):
        pltpu.sync_copy(dstv.at[c], dst_hbm.at[pl.ds(ab + c * 64, 64)])

    for c in range(4):
        for j in range(64):
            idxf = jnp.zeros((16,), jnp.int32) + (c * 64 + j)
            wbuf[j, pl.ds(0, 16)] = plsc.load_gather(wv, [idxf])
        pltpu.async_copy(x_hbm.at[tokv.at[c]], xbuf, sem1).wait()
        pltpu.async_copy(xbuf, xs_hbm.at[dstv.at[c]], sem2).wait()
        pltpu.async_copy(wbuf, ws_hbm.at[dstv.at[c]], sem2).wait()

    @pl.when(wid == 0)
    def _():
        for jv in range(_NTP // 16):
            jvec = (lane + jv * 16) * _TM
            acc = jnp.zeros((16,), jnp.int32)
            for e in range(_E):
                acc = acc + jnp.where(jvec >= start_s[e], 1, 0)
            teidv[pl.ds(jv * 16, 16)] = jnp.minimum(acc - 1, _E - 1)
        pltpu.sync_copy(teidv, teid_hbm)


def _gmm_body(eid_ref, xs_ref, ws_ref, w1_ref, b1_ref, w2_ref, b2_ref, y_ref):
    xb = xs_ref[...].astype(jnp.bfloat16)
    h = jnp.dot(xb, w1_ref[0], preferred_element_type=jnp.float32)
    h = h + b1_ref[0]
    h = 0.5 * h * (1.0 + jax.lax.erf(h / _SQRT2))
    y = jnp.dot(h.astype(jnp.bfloat16), w2_ref[0], preferred_element_type=jnp.float32)
    y_ref[...] = (y + b2_ref[0]) * ws_ref[:, 0:1]


def _gmm(xs, ws, teid, W1b, b1, W2b, b2):
    grid_spec = pltpu.PrefetchScalarGridSpec(
        num_scalar_prefetch=1,
        grid=(_NT,),
        in_specs=[
            pl.BlockSpec((_TM, _DIM), lambda t, eid: (t, 0)),
            pl.BlockSpec((_TM, 128), lambda t, eid: (t, 0)),
            pl.BlockSpec((1, _DIM, _HID), lambda t, eid: (eid[t], 0, 0)),
            pl.BlockSpec((1, 1, _HID), lambda t, eid: (eid[t], 0, 0)),
            pl.BlockSpec((1, _HID, _DIM), lambda t, eid: (eid[t], 0, 0)),
            pl.BlockSpec((1, 1, _DIM), lambda t, eid: (eid[t], 0, 0)),
        ],
        out_specs=pl.BlockSpec((_TM, _DIM), lambda t, eid: (t, 0)),
    )
    return pl.pallas_call(
        _gmm_body,
        grid_spec=grid_spec,
        out_shape=jax.ShapeDtypeStruct((_NPAD, _DIM), jnp.float32),
    )(teid, xs, ws, W1b, b1.reshape(_E, 1, _HID), W2b, b2.reshape(_E, 1, _DIM))


@functools.lru_cache(maxsize=None)
def _combine_kernel():
  return functools.partial(
    pl.kernel,
    out_type=jax.ShapeDtypeStruct((_T, _DIM), jnp.float32),
    mesh=_mesh(),
    scratch_types=[
        pltpu.VMEM((_TPT,), jnp.int32),
        pltpu.VMEM((_TPT,), jnp.int32),
        pltpu.VMEM((32, _DIM), jnp.float32),
        pltpu.VMEM((32, _DIM), jnp.float32),
        pltpu.SemaphoreType.DMA,
    ],
    compiler_params=pltpu.CompilerParams(needs_layout_passes=False),
  )(_combine_body)


def _combine_body(y_hbm, dst_hbm, out_hbm, d0v, d1v, obuf, ybuf, sem):
    wid = lax.axis_index("s") * 2 + lax.axis_index("c")
    tb = wid * _TPT
    pltpu.sync_copy(dst_hbm.at[pl.ds(tb, _TPT)], d0v)
    pltpu.sync_copy(dst_hbm.at[pl.ds(_T + tb, _TPT)], d1v)
    for c in range(_TPT // 32):
        pltpu.async_copy(y_hbm.at[d0v.at[pl.ds(c * 32, 32)]], obuf, sem).wait()
        pltpu.async_copy(y_hbm.at[d1v.at[pl.ds(c * 32, 32)]], ybuf, sem).wait()
        for r in range(32):
            def _addrow(cc, _, r=r):
                off = cc * 16
                obuf[r, pl.ds(off, 16)] = (
                    obuf[r, pl.ds(off, 16)] + ybuf[r, pl.ds(off, 16)]
                )
                return 0
            lax.fori_loop(0, _DIM // 16, _addrow, 0)
        pltpu.sync_copy(obuf, out_hbm.at[pl.ds(tb + c * 32, 32)])


def kernel(x, Wg, bg, W1, b1, W2, b2):
    x2d = x.reshape(_T, _DIM)
    e0, e1, w0, w1 = _gates(x2d, Wg, bg)
    xs, ws, dst, teid = _route_dispatch_kernel()(e0, e1, w0, w1, x2d)
    y = _gmm(xs, ws, teid, W1.astype(jnp.bfloat16), b1,
             W2.astype(jnp.bfloat16), b2)
    out = _combine_kernel()(y, dst)
    return out.reshape(_B, _S, _DIM)


# speedup vs baseline: 5.7727x; 1.0168x over previous
import functools
import math

import jax
import jax.numpy as jnp
from jax import lax
from jax.experimental import pallas as pl
from jax.experimental.pallas import tpu as pltpu
from jax.experimental.pallas import tpu_sc as plsc

_B, _S, _DIM, _HID, _E = 2, 2048, 1024, 4096, 8
_T = _B * _S
_NA = 2 * _T
_EPS = 1e-6
_SQRT2 = math.sqrt(2.0)

_NW = 32
_APT = _NA // _NW
_TPT = _T // _NW
_TM = 256
_NPAD = _NA + _E * _TM
_NT = _NPAD // _TM
_NTP = 48

@functools.lru_cache(maxsize=None)
def _mesh():
    return plsc.VectorSubcoreMesh(
        core_axis_name="c", subcore_axis_name="s", num_cores=2, num_subcores=16
    )


def _gate_body(x_ref, wg_ref, bg_ref, e0_ref, e1_ref, w0_ref, w1_ref):
    logits = jnp.dot(x_ref[...], wg_ref[...], preferred_element_type=jnp.float32)
    logits = logits + bg_ref[...]
    m = jnp.max(logits, axis=-1, keepdims=True)
    ex = jnp.exp(logits - m)
    p = ex / jnp.sum(ex, axis=-1, keepdims=True)
    ii = jax.lax.broadcasted_iota(jnp.int32, p.shape, 1)
    m0 = jnp.max(p, axis=-1, keepdims=True)
    i0 = jnp.min(jnp.where(p == m0, ii, _E), axis=-1, keepdims=True)
    oh0 = ii == i0
    p2 = jnp.where(oh0, -1.0, p)
    m1 = jnp.max(p2, axis=-1, keepdims=True)
    i1 = jnp.min(jnp.where(p2 == m1, ii, _E), axis=-1, keepdims=True)
    oh1 = ii == i1
    s = m0 + m1 + _EPS
    e0_ref[...] = i0[:, 0]
    e1_ref[...] = i1[:, 0]
    w0_ref[...] = (m0 / s)[:, 0]
    w1_ref[...] = (m1 / s)[:, 0]


def _gates(x2d, Wg, bg):
    tt = 1024
    return pl.pallas_call(
        _gate_body,
        grid=(_T // tt,),
        in_specs=[
            pl.BlockSpec((tt, _DIM), lambda t: (t, 0)),
            pl.BlockSpec((_DIM, _E), lambda t: (0, 0)),
            pl.BlockSpec((1, _E), lambda t: (0, 0)),
        ],
        out_specs=[
            pl.BlockSpec((tt,), lambda t: (t,)),
            pl.BlockSpec((tt,), lambda t: (t,)),
            pl.BlockSpec((tt,), lambda t: (t,)),
            pl.BlockSpec((tt,), lambda t: (t,)),
        ],
        out_shape=[
            jax.ShapeDtypeStruct((_T,), jnp.int32),
            jax.ShapeDtypeStruct((_T,), jnp.int32),
            jax.ShapeDtypeStruct((_T,), jnp.float32),
            jax.ShapeDtypeStruct((_T,), jnp.float32),
        ],
    )(x2d, Wg, bg.reshape(1, _E))


@functools.lru_cache(maxsize=None)
def _route_dispatch_kernel():
  return functools.partial(
    pl.kernel,
    out_type=(
        jax.ShapeDtypeStruct((_NPAD, _DIM), jnp.float32),
        jax.ShapeDtypeStruct((_NPAD, 128), jnp.float32),
        jax.ShapeDtypeStruct((_NA,), jnp.int32),
        jax.ShapeDtypeStruct((_NTP,), jnp.int32),
    ),
    mesh=_mesh(),
    scratch_types=[
        pltpu.VMEM((_NA,), jnp.int32),
        pltpu.VMEM((_APT,), jnp.float32),
        pltpu.VMEM((8, 32), jnp.int32),
        pltpu.VMEM((8, 32), jnp.int32),
        pltpu.VMEM((32, _DIM), jnp.float32),
        pltpu.VMEM((32, _DIM), jnp.float32),
        pltpu.VMEM((32, 128), jnp.float32),
        pltpu.VMEM((32, 128), jnp.float32),
        pltpu.VMEM((_NTP,), jnp.int32),
        pltpu.SemaphoreType.DMA,
        pltpu.SemaphoreType.DMA,
    ],
    compiler_params=pltpu.CompilerParams(needs_layout_passes=False),
  )(_route_dispatch_body)


def _route_dispatch_body(e0_hbm, e1_hbm, w0_hbm, w1_hbm, x_hbm,
                    xs_hbm, ws_hbm, dst_hbm, teid_hbm,
                    evall, wv, dstv, tokv, xbuf0, xbuf1, wbuf0, wbuf1,
                    teidv, sem1, sem2):
    wid = lax.axis_index("s") * 2 + lax.axis_index("c")
    ab = wid * _APT
    lane = lax.iota(jnp.int32, 16)

    pltpu.sync_copy(e0_hbm, evall.at[pl.ds(0, _T)])
    pltpu.sync_copy(e1_hbm, evall.at[pl.ds(_T, _T)])

    @pl.when(wid < 16)
    def _():
        pltpu.sync_copy(w0_hbm.at[pl.ds(ab, _APT)], wv)

    @pl.when(wid >= 16)
    def _():
        pltpu.sync_copy(w1_hbm.at[pl.ds(ab - _T, _APT)], wv)

    lim = ab // 16
    zz = jnp.zeros((16,), jnp.int32)

    def _hist(j, carry):
        accs = list(carry)
        v = evall[pl.ds(j * 16, 16)]
        sel = jnp.where(j < lim, 1, 0)
        for e in range(_E):
            mme = jnp.where(v == e, 1, 0)
            accs[e] = accs[e] + mme
            accs[_E + e] = accs[_E + e] + mme * sel
        return tuple(accs)

    accs = lax.fori_loop(0, _NA // 16, _hist, (zz,) * (2 * _E))
    total = zz
    basem = zz
    for e in range(_E):
        total = total + jnp.where(lane == e, jnp.sum(accs[e], axis=0), 0)
        basem = basem + jnp.where(lane == e, jnp.sum(accs[_E + e], axis=0), 0)
    cap = ((total + (_TM - 1)) >> 8) << 8
    cum = plsc.cumsum(cap)
    start = cum - cap
    basem = basem + start
    base_s = [jnp.max(jnp.where(lane == e, basem, 0), axis=0) for e in range(_E)]
    start_s = [jnp.max(jnp.where(lane == e, start, 0), axis=0) for e in range(_E)]

    for i in range(_APT // 16):
        v = evall[pl.ds(ab + i * 16, 16)]
        acc = jnp.zeros((16,), jnp.int32)
        for e in range(_E):
            mm = v == e
            ones = jnp.where(mm, 1, 0)
            r = plsc.cumsum(ones)
            acc = jnp.where(mm, base_s[e] + r - 1, acc)
            base_s[e] = base_s[e] + jnp.max(r, axis=0)
        dstv[i // 2, pl.ds((i % 2) * 16, 16)] = jnp.clip(acc, 0, _NPAD - 1)
        tok = lane + (ab + i * 16) - jnp.where(wid >= 16, _T, 0)
        tokv[i // 2, pl.ds((i % 2) * 16, 16)] = jnp.clip(tok, 0, _T - 1)

    for c in range(8):
        pltpu.sync_copy(dstv.at[c], dst_hbm.at[pl.ds(ab + c * 32, 32)])

    xb = (xbuf0, xbuf1)
    wb = (wbuf0, wbuf1)
    pend = []
    gd = pltpu.async_copy(x_hbm.at[tokv.at[0]], xb[0], sem1)
    for c in range(8):
        b = c % 2
        if c + 1 < 8:
            if c >= 1:
                for d in pend:
                    d.wait()
                pend = []
            gd_next = pltpu.async_copy(x_hbm.at[tokv.at[c + 1]], xb[1 - b], sem1)
        gd.wait()
        for j in range(32):
            idxf = jnp.zeros((16,), jnp.int32) + (c * 32 + j)
            wb[b][j, pl.ds(0, 16)] = plsc.load_gather(wv, [idxf])
        pend.append(pltpu.async_copy(xb[b], xs_hbm.at[dstv.at[c]], sem2))
        pend.append(pltpu.async_copy(wb[b], ws_hbm.at[dstv.at[c]], sem2))
        if c + 1 < 8:
            gd = gd_next
    for d in pend:
        d.wait()

    @pl.when(wid == 0)
    def _():
        for jv in range(_NTP // 16):
            jvec = (lane + jv * 16) * _TM
            acc = jnp.zeros((16,), jnp.int32)
            for e in range(_E):
                acc = acc + jnp.where(jvec >= start_s[e], 1, 0)
            teidv[pl.ds(jv * 16, 16)] = jnp.minimum(acc - 1, _E - 1)
        pltpu.sync_copy(teidv, teid_hbm)


def _gmm_body(eid_ref, xs_ref, ws_ref, w1_ref, b1_ref, w2_ref, b2_ref, y_ref):
    xb = xs_ref[...].astype(jnp.bfloat16)
    h = jnp.dot(xb, w1_ref[0], preferred_element_type=jnp.float32)
    h = h + b1_ref[0]
    h = 0.5 * h * (1.0 + jax.lax.erf(h / _SQRT2))
    y = jnp.dot(h.astype(jnp.bfloat16), w2_ref[0], preferred_element_type=jnp.float32)
    y_ref[...] = (y + b2_ref[0]) * ws_ref[:, 0:1]


def _gmm(xs, ws, teid, W1b, b1, W2b, b2):
    grid_spec = pltpu.PrefetchScalarGridSpec(
        num_scalar_prefetch=1,
        grid=(_NT,),
        in_specs=[
            pl.BlockSpec((_TM, _DIM), lambda t, eid: (t, 0)),
            pl.BlockSpec((_TM, 128), lambda t, eid: (t, 0)),
            pl.BlockSpec((1, _DIM, _HID), lambda t, eid: (eid[t], 0, 0)),
            pl.BlockSpec((1, 1, _HID), lambda t, eid: (eid[t], 0, 0)),
            pl.BlockSpec((1, _HID, _DIM), lambda t, eid: (eid[t], 0, 0)),
            pl.BlockSpec((1, 1, _DIM), lambda t, eid: (eid[t], 0, 0)),
        ],
        out_specs=pl.BlockSpec((_TM, _DIM), lambda t, eid: (t, 0)),
    )
    return pl.pallas_call(
        _gmm_body,
        grid_spec=grid_spec,
        out_shape=jax.ShapeDtypeStruct((_NPAD, _DIM), jnp.float32),
    )(teid, xs, ws, W1b, b1.reshape(_E, 1, _HID), W2b, b2.reshape(_E, 1, _DIM))


@functools.lru_cache(maxsize=None)
def _combine_kernel():
  return functools.partial(
    pl.kernel,
    out_type=jax.ShapeDtypeStruct((_T, _DIM), jnp.float32),
    mesh=_mesh(),
    scratch_types=[
        pltpu.VMEM((_TPT,), jnp.int32),
        pltpu.VMEM((_TPT,), jnp.int32),
        pltpu.VMEM((32, _DIM), jnp.float32),
        pltpu.VMEM((32, _DIM), jnp.float32),
        pltpu.SemaphoreType.DMA,
    ],
    compiler_params=pltpu.CompilerParams(needs_layout_passes=False),
  )(_combine_body)


def _combine_body(y_hbm, dst_hbm, out_hbm, d0v, d1v, obuf, ybuf, sem):
    wid = lax.axis_index("s") * 2 + lax.axis_index("c")
    tb = wid * _TPT
    pltpu.sync_copy(dst_hbm.at[pl.ds(tb, _TPT)], d0v)
    pltpu.sync_copy(dst_hbm.at[pl.ds(_T + tb, _TPT)], d1v)
    for c in range(_TPT // 32):
        pltpu.async_copy(y_hbm.at[d0v.at[pl.ds(c * 32, 32)]], obuf, sem).wait()
        pltpu.async_copy(y_hbm.at[d1v.at[pl.ds(c * 32, 32)]], ybuf, sem).wait()
        for r in range(32):
            def _addrow(cc, _, r=r):
                off = cc * 64
                for k in range(4):
                    o = off + k * 16
                    obuf[r, pl.ds(o, 16)] = (
                        obuf[r, pl.ds(o, 16)] + ybuf[r, pl.ds(o, 16)]
                    )
                return 0
            lax.fori_loop(0, _DIM // 64, _addrow, 0)
        pltpu.sync_copy(obuf, out_hbm.at[pl.ds(tb + c * 32, 32)])


def kernel(x, Wg, bg, W1, b1, W2, b2):
    x2d = x.reshape(_T, _DIM)
    e0, e1, w0, w1 = _gates(x2d, Wg, bg)
    xs, ws, dst, teid = _route_dispatch_kernel()(e0, e1, w0, w1, x2d)
    y = _gmm(xs, ws, teid, W1.astype(jnp.bfloat16), b1,
             W2.astype(jnp.bfloat16), b2)
    out = _combine_kernel()(y, dst)
    return out.reshape(_B, _S, _DIM)
